# Initial kernel scaffold; baseline (speedup 1.0000x reference)
import jax
import jax.numpy as jnp
from jax.experimental import pallas as pl


def kernel(te, t):
    raise NotImplementedError("write your pallas kernel here")


# speedup vs baseline: 2.2642x; 2.2642x over previous
import functools

import jax
import jax.numpy as jnp
from jax import lax
from jax.experimental import pallas as pl
from jax.experimental.pallas import tpu as pltpu
from jax.experimental.pallas import tpu_sc as plsc

_D = 128
_B = 16384
_NC = 2
_NS = 16
_NW = _NC * _NS
_BPW = _B // _NW
_CHUNK = 128
_NCHUNK = _BPW // _CHUNK

_mesh = plsc.VectorSubcoreMesh(core_axis_name="c", subcore_axis_name="s")


@functools.partial(
    pl.kernel,
    mesh=_mesh,
    out_type=jax.ShapeDtypeStruct((_B, _D), jnp.float32),
    scratch_types=[
        pltpu.VMEM((_BPW,), jnp.int32),
        pltpu.VMEM((_BPW, _D), jnp.float32),
        pltpu.SemaphoreType.DMA,
    ],
)
def _lookup(te_hbm, t_hbm, out_hbm, idx_v, rows_v, sem):
    wid = lax.axis_index("s") * _NC + lax.axis_index("c")
    base = wid * _BPW
    pltpu.sync_copy(t_hbm.at[pl.ds(base, _BPW)], idx_v)
    copies = []
    for j in range(_NCHUNK):
        copies.append(
            pltpu.async_copy(
                te_hbm.at[idx_v.at[pl.ds(j * _CHUNK, _CHUNK)]],
                rows_v.at[pl.ds(j * _CHUNK, _CHUNK)],
                sem,
            )
        )
    for c in copies:
        c.wait()
    pltpu.sync_copy(rows_v, out_hbm.at[pl.ds(base, _BPW)])


def kernel(te, t):
    return _lookup(te, t.astype(jnp.int32))


# speedup vs baseline: 2.6532x; 1.1718x over previous
import functools

import jax
import jax.numpy as jnp
from jax import lax
from jax.experimental import pallas as pl
from jax.experimental.pallas import tpu as pltpu
from jax.experimental.pallas import tpu_sc as plsc

_T = 1000
_D = 128
_B = 16384
_NC = 2
_NS = 16
_NW = _NC * _NS
_BPW = _B // _NW
_CHUNK = 128
_NCHUNK = _BPW // _CHUNK

_mesh = plsc.VectorSubcoreMesh(core_axis_name="c", subcore_axis_name="s")


@functools.partial(
    pl.kernel,
    mesh=_mesh,
    out_type=jax.ShapeDtypeStruct((_B, _D), jnp.float32),
    scratch_types=[
        pltpu.VMEM((_BPW,), jnp.int32),
        pltpu.VMEM((_BPW, _D), jnp.float32),
        pltpu.VMEM_SHARED((_T, _D), jnp.float32),
        pltpu.SemaphoreType.DMA,
    ],
)
def _lookup(te_hbm, t_hbm, out_hbm, idx_v, rows_v, table_s, sem):
    sid = lax.axis_index("s")
    wid = sid * _NC + lax.axis_index("c")
    base = wid * _BPW

    @pl.when(sid == 0)
    def _():
        pltpu.sync_copy(te_hbm, table_s)

    pltpu.sync_copy(t_hbm.at[pl.ds(base, _BPW)], idx_v)
    plsc.subcore_barrier()

    copies = []
    for j in range(_NCHUNK):
        copies.append(
            pltpu.async_copy(
                table_s.at[idx_v.at[pl.ds(j * _CHUNK, _CHUNK)]],
                rows_v.at[pl.ds(j * _CHUNK, _CHUNK)],
                sem,
            )
        )
    for c in copies:
        c.wait()
    pltpu.sync_copy(rows_v, out_hbm.at[pl.ds(base, _BPW)])


def kernel(te, t):
    return _lookup(te, t.astype(jnp.int32))


# speedup vs baseline: 2.7559x; 1.0387x over previous
import functools

import jax
import jax.numpy as jnp
from jax import lax
from jax.experimental import pallas as pl
from jax.experimental.pallas import tpu as pltpu
from jax.experimental.pallas import tpu_sc as plsc

_T = 1000
_D = 128
_B = 16384
_NC = 2
_NS = 16
_NW = _NC * _NS
_BPW = _B // _NW
_CHUNK = 128
_NCHUNK = _BPW // _CHUNK
_STAGERS = 8
_STAGE_SPLIT = [(k * 128, 128) for k in range(7)] + [(896, 104)]

_mesh = plsc.VectorSubcoreMesh(core_axis_name="c", subcore_axis_name="s")


@functools.partial(
    pl.kernel,
    mesh=_mesh,
    out_type=jax.ShapeDtypeStruct((_B, _D), jnp.float32),
    scratch_types=[
        pltpu.VMEM((_BPW,), jnp.int32),
        pltpu.VMEM((_BPW, _D), jnp.float32),
        pltpu.VMEM_SHARED((_T, _D), jnp.float32),
        pltpu.SemaphoreType.DMA,
        pltpu.SemaphoreType.DMA,
    ],
)
def _lookup(te_hbm, t_hbm, out_hbm, idx_v, rows_v, table_s, gsem, wsem):
    sid = lax.axis_index("s")
    wid = sid * _NC + lax.axis_index("c")
    base = wid * _BPW

    for k in range(_STAGERS):
        @pl.when(sid == k)
        def _(k=k):
            r0, nrows = _STAGE_SPLIT[k]
            pltpu.sync_copy(
                te_hbm.at[pl.ds(r0, nrows)],
                table_s.at[pl.ds(r0, nrows)],
            )

    pltpu.sync_copy(t_hbm.at[pl.ds(base, _BPW)], idx_v)
    plsc.subcore_barrier()

    gathers = []
    for j in range(_NCHUNK):
        gathers.append(
            pltpu.async_copy(
                table_s.at[idx_v.at[pl.ds(j * _CHUNK, _CHUNK)]],
                rows_v.at[pl.ds(j * _CHUNK, _CHUNK)],
                gsem,
            )
        )
    writes = []
    for j in range(_NCHUNK):
        gathers[j].wait()
        writes.append(
            pltpu.async_copy(
                rows_v.at[pl.ds(j * _CHUNK, _CHUNK)],
                out_hbm.at[pl.ds(base + j * _CHUNK, _CHUNK)],
                wsem,
            )
        )
    for c in writes:
        c.wait()


def kernel(te, t):
    return _lookup(te, t.astype(jnp.int32))
